# baseline (speedup 1.0000x reference)
import functools

import jax
import jax.numpy as jnp
from jax.experimental import pallas as pl
from jax.experimental.pallas import tpu as pltpu


def _enc_kernel(terms_sm, cnts_ref, tab_hbm, out_ref, rows, sem, *, BH, L, D):
    j = pl.program_id(0)
    M = BH * L

    for t in range(M):
        idx = terms_sm[j * BH + t // L, t % L]
        pltpu.make_async_copy(
            tab_hbm.at[pl.ds(idx, 1), :],
            rows.at[pl.ds(t, 1), :],
            sem,
        ).start()

    logc = jnp.log(cnts_ref[0].reshape(BH, L, 1))

    pltpu.make_async_copy(
        tab_hbm.at[pl.ds(0, M), :], rows.at[pl.ds(0, M), :], sem,
    ).wait()

    G = rows[...].reshape(BH, L, 128)
    w_tok = G[:, :, D:D + 1]
    logits = w_tok + logc
    m = jnp.max(logits, axis=1, keepdims=True)
    e = jnp.exp(logits - m)
    s = jnp.sum(e, axis=1, keepdims=True)
    p = e / s

    out_ref[0] = jnp.sum(p * G, axis=1)


def kernel(terms, cnts, weights_table, embed_table):
    B, L = terms.shape
    V, D = embed_table.shape
    BH = B // 2

    lane = jax.lax.broadcasted_iota(jnp.int32, (V, 128), 1)
    fused = jnp.where(
        lane == D,
        weights_table.astype(jnp.float32),
        jnp.pad(embed_table.astype(jnp.float32), ((0, 0), (0, 128 - D))),
    )

    c3 = cnts.astype(jnp.float32).reshape(2, BH * L, 1)

    kernel_fn = functools.partial(_enc_kernel, BH=BH, L=L, D=D)

    out = pl.pallas_call(
        kernel_fn,
        out_shape=jax.ShapeDtypeStruct((2, BH, 128), jnp.float32),
        grid_spec=pltpu.PrefetchScalarGridSpec(
            num_scalar_prefetch=1,
            grid=(2,),
            in_specs=[
                pl.BlockSpec((1, BH * L, 1), lambda j, t: (j, 0, 0)),
                pl.BlockSpec(memory_space=pl.ANY),
            ],
            out_specs=pl.BlockSpec((1, BH, 128), lambda j, t: (j, 0, 0)),
            scratch_shapes=[
                pltpu.VMEM((BH * L, 128), jnp.float32),
                pltpu.SemaphoreType.DMA,
            ],
        ),
        compiler_params=pltpu.CompilerParams(
            dimension_semantics=("parallel",),
            vmem_limit_bytes=32 * 1024 * 1024,
        ),
    )(terms.astype(jnp.int32), c3, fused)

    return out.reshape(B, 128)[:, :D]
